# baseline (speedup 1.0000x reference)
import functools
import math

import jax
import jax.numpy as jnp
from jax import lax
from jax.experimental import pallas as pl
from jax.experimental.pallas import tpu as pltpu
from jax.experimental.pallas import tpu_sc as plsc

_LANES = 16
_NG = 2
_RBC = 2
_NO = 2


@functools.partial(jax.jit, static_argnames=("d_model", "scale"))
def _gather_scale(idx2d, table, d_model, scale):
    nw, rows_per_w = idx2d.shape
    info = plsc.get_sparse_core_info()
    nc = info.num_cores
    mesh = plsc.VectorSubcoreMesh(core_axis_name="c", subcore_axis_name="s")
    seq = 200
    rb_per_w = rows_per_w // seq
    n_batch = nw * rb_per_w
    nch = rb_per_w // _RBC
    assert nch % _NG == 0 and nch // _NG >= 3

    @functools.partial(
        pl.kernel,
        mesh=mesh,
        out_type=jax.ShapeDtypeStruct((n_batch, seq, 2 * d_model), jnp.float32),
        scratch_types=[
            pltpu.VMEM((rows_per_w,), jnp.int32),
            pltpu.VMEM((_NG, _RBC * seq, d_model), jnp.float32),
            pltpu.VMEM((_NO, _RBC, seq, d_model), jnp.float32),
            pltpu.SemaphoreType.DMA((_NG,)),
            pltpu.SemaphoreType.DMA((_NO,)),
        ],
        compiler_params=pltpu.CompilerParams(use_tc_tiling_on_sc=False),
    )
    def k(idx_hbm, table_hbm, out_hbm, idx_v, g, o, g_sem, o_sem):
        wid = lax.axis_index("s") * nc + lax.axis_index("c")
        row0 = wid * rb_per_w
        pltpu.sync_copy(idx_hbm.at[wid], idx_v)

        def start_gather(j, b):
            pltpu.async_copy(
                table_hbm.at[idx_v.at[pl.ds(j * (_RBC * seq), _RBC * seq)]],
                g.at[b],
                g_sem.at[b],
            )

        def wait_gather(j, b):
            pltpu.make_async_copy(
                table_hbm.at[idx_v.at[pl.ds(j * (_RBC * seq), _RBC * seq)]],
                g.at[b],
                g_sem.at[b],
            ).wait()

        def scale_chunk(b, p):
            def srow(t, c2):
                for q in range(_RBC):
                    for d in range(d_model // _LANES):
                        s = pl.ds(d * _LANES, _LANES)
                        o[p, q, t, s] = g[b, q * seq + t, s] * scale
                return c2

            lax.fori_loop(0, seq, srow, 0, unroll=4)

        def start_out(j, p):
            pltpu.async_copy(
                o.at[p],
                out_hbm.at[pl.ds(row0 + j * _RBC, _RBC), slice(None), pl.ds(0, d_model)],
                o_sem.at[p],
            )

        def wait_out(p):
            pltpu.make_async_copy(
                o.at[p],
                out_hbm.at[pl.ds(row0, _RBC), slice(None), pl.ds(0, d_model)],
                o_sem.at[p],
            ).wait()

        for b in range(_NG):
            start_gather(b, b)

        for b in range(_NG):
            wait_gather(b, b)
            if b >= _NO:
                wait_out(b % _NO)
            scale_chunk(b, b % _NO)
            start_out(b, b % _NO)
            start_gather(b + _NG, b)

        n_groups = nch // _NG

        def group(gr, carry):
            for b in range(_NG):
                j = gr * _NG + b
                wait_gather(j, b)
                wait_out(b % _NO)
                scale_chunk(b, b % _NO)
                start_out(j, b % _NO)
                start_gather(j + _NG, b)
            return carry

        lax.fori_loop(1, n_groups - 1, group, 0)

        for b in range(_NG):
            j = (n_groups - 1) * _NG + b
            wait_gather(j, b)
            wait_out(b % _NO)
            scale_chunk(b, b % _NO)
            start_out(j, b % _NO)

        for p in range(_NO):
            wait_out(p)

    return k(idx2d, table)


def kernel(x, table):
    d_model = table.shape[1]
    n_rows = x.size
    scale = math.sqrt(d_model)
    info = plsc.get_sparse_core_info()
    nw = info.num_cores * info.num_subcores
    idx2d = x.reshape(-1).astype(jnp.int32).reshape(nw, n_rows // nw)
    out_pad = _gather_scale(idx2d, table, d_model, scale)
    return out_pad[:, :, :d_model]


# speedup vs baseline: 1.0364x; 1.0364x over previous
import functools
import math

import jax
import jax.numpy as jnp
from jax import lax
from jax.experimental import pallas as pl
from jax.experimental.pallas import tpu as pltpu
from jax.experimental.pallas import tpu_sc as plsc

_LANES = 16
_NG = 4
_NO = 2


@functools.partial(jax.jit, static_argnames=("d_model", "scale"))
def _gather_scale(idx2d, table, d_model, scale):
    nw, rows_per_w = idx2d.shape
    info = plsc.get_sparse_core_info()
    nc = info.num_cores
    mesh = plsc.VectorSubcoreMesh(core_axis_name="c", subcore_axis_name="s")
    seq = 200
    rb_per_w = rows_per_w // seq
    n_batch = nw * rb_per_w
    nch = rb_per_w
    assert nch % _NG == 0 and nch // _NG >= 3

    @functools.partial(
        pl.kernel,
        mesh=mesh,
        out_type=jax.ShapeDtypeStruct((n_batch, seq, 2 * d_model), jnp.float32),
        scratch_types=[
            pltpu.VMEM((rows_per_w,), jnp.int32),
            pltpu.VMEM((_NG, seq, d_model), jnp.float32),
            pltpu.VMEM((_NO, 1, seq, d_model), jnp.float32),
            pltpu.SemaphoreType.DMA((_NG,)),
            pltpu.SemaphoreType.DMA((_NO,)),
        ],
        compiler_params=pltpu.CompilerParams(use_tc_tiling_on_sc=False),
    )
    def k(idx_hbm, table_hbm, out_hbm, idx_v, g, o, g_sem, o_sem):
        wid = lax.axis_index("s") * nc + lax.axis_index("c")
        row0 = wid * rb_per_w
        pltpu.sync_copy(idx_hbm.at[wid], idx_v)

        def start_gather(j, b):
            pltpu.async_copy(
                table_hbm.at[idx_v.at[pl.ds(j * seq, seq)]], g.at[b], g_sem.at[b]
            )

        def wait_gather(j, b):
            pltpu.make_async_copy(
                table_hbm.at[idx_v.at[pl.ds(j * seq, seq)]], g.at[b], g_sem.at[b]
            ).wait()

        def scale_chunk(b, p):
            def srow(t, c2):
                for d in range(d_model // _LANES):
                    s = pl.ds(d * _LANES, _LANES)
                    o[p, 0, t, s] = g[b, t, s] * scale
                return c2

            lax.fori_loop(0, seq, srow, 0, unroll=8)

        def start_out(j, p):
            pltpu.async_copy(
                o.at[p],
                out_hbm.at[pl.ds(row0 + j, 1), slice(None), pl.ds(0, d_model)],
                o_sem.at[p],
            )

        def wait_out(p):
            pltpu.make_async_copy(
                o.at[p],
                out_hbm.at[pl.ds(row0, 1), slice(None), pl.ds(0, d_model)],
                o_sem.at[p],
            ).wait()

        for b in range(_NG):
            start_gather(b, b)

        for b in range(_NG):
            wait_gather(b, b)
            if b >= _NO:
                wait_out(b % _NO)
            scale_chunk(b, b % _NO)
            start_out(b, b % _NO)
            start_gather(b + _NG, b)

        n_groups = nch // _NG

        def group(gr, carry):
            for b in range(_NG):
                j = gr * _NG + b
                wait_gather(j, b)
                wait_out(b % _NO)
                scale_chunk(b, b % _NO)
                start_out(j, b % _NO)
                start_gather(j + _NG, b)
            return carry

        lax.fori_loop(1, n_groups - 1, group, 0)

        for b in range(_NG):
            j = (n_groups - 1) * _NG + b
            wait_gather(j, b)
            wait_out(b % _NO)
            scale_chunk(b, b % _NO)
            start_out(j, b % _NO)

        for p in range(_NO):
            wait_out(p)

    return k(idx2d, table)


def kernel(x, table):
    d_model = table.shape[1]
    n_rows = x.size
    scale = math.sqrt(d_model)
    info = plsc.get_sparse_core_info()
    nw = info.num_cores * info.num_subcores
    idx2d = x.reshape(-1).astype(jnp.int32).reshape(nw, n_rows // nw)
    out_pad = _gather_scale(idx2d, table, d_model, scale)
    return out_pad[:, :, :d_model]


# speedup vs baseline: 1.0397x; 1.0032x over previous
import functools
import math

import jax
import jax.numpy as jnp
from jax import lax
from jax.experimental import pallas as pl
from jax.experimental.pallas import tpu as pltpu
from jax.experimental.pallas import tpu_sc as plsc

_LANES = 16
_NG = 4
_NO = 2


@functools.partial(jax.jit, static_argnames=("d_model", "scale"))
def _gather_scale(idx2d, table, d_model, scale):
    nw, rows_per_w = idx2d.shape
    info = plsc.get_sparse_core_info()
    nc = info.num_cores
    mesh = plsc.VectorSubcoreMesh(core_axis_name="c", subcore_axis_name="s")
    seq = 200
    rb_per_w = rows_per_w // seq
    n_batch = nw * rb_per_w
    nch = rb_per_w
    assert nch % _NG == 0 and nch // _NG >= 3

    @functools.partial(
        pl.kernel,
        mesh=mesh,
        out_type=jax.ShapeDtypeStruct((n_batch, seq, 2 * d_model), jnp.float32),
        scratch_types=[
            pltpu.VMEM((rows_per_w,), jnp.int32),
            pltpu.VMEM((_NG, seq, d_model), jnp.float32),
            pltpu.VMEM((_NO, 1, seq, d_model), jnp.float32),
            pltpu.SemaphoreType.DMA((_NG,)),
            pltpu.SemaphoreType.DMA((_NO,)),
        ],
        compiler_params=pltpu.CompilerParams(use_tc_tiling_on_sc=False),
    )
    def k(idx_hbm, table_hbm, out_hbm, idx_v, g, o, g_sem, o_sem):
        wid = lax.axis_index("s") * nc + lax.axis_index("c")
        row0 = wid * rb_per_w
        pltpu.sync_copy(idx_hbm.at[wid], idx_v)

        def start_gather(j, b):
            pltpu.async_copy(
                table_hbm.at[idx_v.at[pl.ds(j * seq, seq)]], g.at[b], g_sem.at[b]
            )

        def wait_gather(j, b):
            pltpu.make_async_copy(
                table_hbm.at[idx_v.at[pl.ds(j * seq, seq)]], g.at[b], g_sem.at[b]
            ).wait()

        def scale_chunk(b, p):
            def srow(t, c2):
                for d in range(d_model // _LANES):
                    s = pl.ds(d * _LANES, _LANES)
                    o[p, 0, t, s] = g[b, t, s] * scale
                return c2

            lax.fori_loop(0, seq, srow, 0, unroll=16)

        def start_out(j, p):
            pltpu.async_copy(
                o.at[p],
                out_hbm.at[pl.ds(row0 + j, 1), slice(None), pl.ds(0, d_model)],
                o_sem.at[p],
            )

        def wait_out(p):
            pltpu.make_async_copy(
                o.at[p],
                out_hbm.at[pl.ds(row0, 1), slice(None), pl.ds(0, d_model)],
                o_sem.at[p],
            ).wait()

        for b in range(_NG):
            start_gather(b, b)

        for b in range(_NG):
            wait_gather(b, b)
            if b >= _NO:
                wait_out(b % _NO)
            scale_chunk(b, b % _NO)
            start_out(b, b % _NO)
            start_gather(b + _NG, b)

        n_groups = nch // _NG

        def group(gr, carry):
            for b in range(_NG):
                j = gr * _NG + b
                wait_gather(j, b)
                wait_out(b % _NO)
                scale_chunk(b, b % _NO)
                start_out(j, b % _NO)
                start_gather(j + _NG, b)
            return carry

        lax.fori_loop(1, n_groups - 1, group, 0)

        for b in range(_NG):
            j = (n_groups - 1) * _NG + b
            wait_gather(j, b)
            wait_out(b % _NO)
            scale_chunk(b, b % _NO)
            start_out(j, b % _NO)

        for p in range(_NO):
            wait_out(p)

    return k(idx2d, table)


def kernel(x, table):
    d_model = table.shape[1]
    n_rows = x.size
    scale = math.sqrt(d_model)
    info = plsc.get_sparse_core_info()
    nw = info.num_cores * info.num_subcores
    idx2d = x.reshape(-1).astype(jnp.int32).reshape(nw, n_rows // nw)
    out_pad = _gather_scale(idx2d, table, d_model, scale)
    return out_pad[:, :, :d_model]


# speedup vs baseline: 1.1710x; 1.1263x over previous
import functools
import math

import jax
import jax.numpy as jnp
from jax import lax
from jax.experimental import pallas as pl
from jax.experimental.pallas import tpu as pltpu
from jax.experimental.pallas import tpu_sc as plsc

_LANES = 16
_NG = 4
_NO = 2


@functools.partial(jax.jit, static_argnames=("d_model", "scale"))
def _gather_scale(idx2d, table, d_model, scale):
    nw, rows_per_w = idx2d.shape
    info = plsc.get_sparse_core_info()
    nc = info.num_cores
    mesh = plsc.VectorSubcoreMesh(core_axis_name="c", subcore_axis_name="s")
    seq = 200
    rb_per_w = rows_per_w // seq
    n_batch = nw * rb_per_w
    nch = rb_per_w
    assert nch % _NG == 0 and nch // _NG >= 3

    @functools.partial(
        pl.kernel,
        mesh=mesh,
        out_type=jax.ShapeDtypeStruct((n_batch, seq, 2 * d_model), jnp.float32),
        scratch_types=[
            pltpu.VMEM((rows_per_w,), jnp.int32),
            pltpu.VMEM((_NG, seq, d_model), jnp.float32),
            pltpu.VMEM((_NO, 1, seq, d_model), jnp.float32),
            pltpu.SemaphoreType.DMA((_NG,)),
            pltpu.SemaphoreType.DMA((_NO,)),
        ],
        compiler_params=pltpu.CompilerParams(use_tc_tiling_on_sc=False),
    )
    def k(idx_hbm, table_hbm, out_hbm, idx_v, g, o, g_sem, o_sem):
        wid = lax.axis_index("s") * nc + lax.axis_index("c")
        row0 = wid * rb_per_w
        pltpu.sync_copy(idx_hbm.at[wid], idx_v)

        def start_gather(j, b):
            pltpu.async_copy(
                table_hbm.at[idx_v.at[pl.ds(j * seq, seq)]], g.at[b], g_sem.at[b]
            )

        def wait_gather(j, b):
            pltpu.make_async_copy(
                table_hbm.at[idx_v.at[pl.ds(j * seq, seq)]], g.at[b], g_sem.at[b]
            ).wait()

        def scale_chunk(b, p):
            def srow(t, c2):
                for d in range(d_model // _LANES):
                    s = pl.ds(d * _LANES, _LANES)
                    o[p, 0, t, s] = g[b, t, s] * scale
                return c2

            lax.fori_loop(0, seq, srow, 0, unroll=25)

        def start_out(j, p):
            pltpu.async_copy(
                o.at[p],
                out_hbm.at[pl.ds(row0 + j, 1), slice(None), pl.ds(0, d_model)],
                o_sem.at[p],
            )

        def wait_out(p):
            pltpu.make_async_copy(
                o.at[p],
                out_hbm.at[pl.ds(row0, 1), slice(None), pl.ds(0, d_model)],
                o_sem.at[p],
            ).wait()

        for b in range(_NG):
            start_gather(b, b)

        for b in range(_NG):
            wait_gather(b, b)
            if b >= _NO:
                wait_out(b % _NO)
            scale_chunk(b, b % _NO)
            start_out(b, b % _NO)
            start_gather(b + _NG, b)

        n_groups = nch // _NG

        def group(gr, carry):
            for b in range(_NG):
                j = gr * _NG + b
                wait_gather(j, b)
                wait_out(b % _NO)
                scale_chunk(b, b % _NO)
                start_out(j, b % _NO)
                start_gather(j + _NG, b)
            return carry

        lax.fori_loop(1, n_groups - 1, group, 0)

        for b in range(_NG):
            j = (n_groups - 1) * _NG + b
            wait_gather(j, b)
            wait_out(b % _NO)
            scale_chunk(b, b % _NO)
            start_out(j, b % _NO)

        for p in range(_NO):
            wait_out(p)

    return k(idx2d, table)


def kernel(x, table):
    d_model = table.shape[1]
    n_rows = x.size
    scale = math.sqrt(d_model)
    info = plsc.get_sparse_core_info()
    nw = info.num_cores * info.num_subcores
    idx2d = x.reshape(-1).astype(jnp.int32).reshape(nw, n_rows // nw)
    out_pad = _gather_scale(idx2d, table, d_model, scale)
    return out_pad[:, :, :d_model]
